# baseline (speedup 1.0000x reference)
import functools

import jax
import jax.numpy as jnp
from jax import lax
from jax.experimental import pallas as pl
from jax.experimental.pallas import tpu as pltpu
from jax.experimental.pallas import tpu_sc as plsc

NP_ = 20000
NT_ = 80000
N = NP_ + NT_
D = 32
E = 1600000
HALF = N // 2

NS = 16
WIN_E = 256
WIN_R = WIN_E // 128
NWIN = 396
EPT = WIN_E * NWIN
E_PAD = NS * EPT
NWTOT = NS * NWIN

TRASH_MASK = 2047
ACC_ROWS = 52224
SLAB = 3128
SLAB_LAST = HALF - 15 * SLAB
ZSLAB = ACC_ROWS // NS
RBUF = 2 * WIN_E


def _step_body(emb, cwin, out, cb, rows, acc, semc, semg, sems):
    c = lax.axis_index("c")
    s = lax.axis_index("s")
    win0 = s * NWIN
    base_node = c * HALF
    zvec = jnp.zeros((16,), jnp.float32)

    def _zrow(r, carry):
        rows[r, 0:16] = zvec
        rows[r, 16:32] = zvec
        return carry

    lax.fori_loop(0, RBUF, _zrow, 0, unroll=4)
    zbase = pl.multiple_of(s * ZSLAB, 8)
    for i in range(6):
        pltpu.sync_copy(rows.at[pl.ds(0, RBUF)],
                        acc.at[pl.ds(zbase + i * RBUF, RBUF)])
    pltpu.sync_copy(rows.at[pl.ds(0, 192)],
                    acc.at[pl.ds(zbase + 6 * RBUF, 192)])
    plsc.subcore_barrier()

    def _fire_gathers(cs, rs):
        for k in range(WIN_R):
            pltpu.async_copy(emb.at[cb.at[cs, k]],
                             rows.at[pl.ds(rs * WIN_E + k * 128, 128)], semg)

    def _drain_gathers(cs, rs):
        for k in range(WIN_R):
            pltpu.make_async_copy(
                emb.at[cb.at[cs, k]],
                rows.at[pl.ds(rs * WIN_E + k * 128, 128)], semg).wait()

    def _fire_scatters(cs, rs):
        for k in range(WIN_R):
            pltpu.async_copy(rows.at[pl.ds(rs * WIN_E + k * 128, 128)],
                             acc.at[cb.at[cs, WIN_R + k]], sems, add=True)

    def _drain_scatters(cs, rs):
        for k in range(WIN_R):
            pltpu.make_async_copy(
                rows.at[pl.ds(rs * WIN_E + k * 128, 128)],
                acc.at[cb.at[cs, WIN_R + k]], sems).wait()

    d0 = pltpu.async_copy(cwin.at[win0], cb.at[0], semc)
    pltpu.async_copy(cwin.at[win0 + 1], cb.at[1], semc)
    d0.wait()
    _fire_gathers(0, 0)

    def _super(jj, carry):
        i0 = jj * 6
        for u in range(6):
            i = i0 + u
            cs = u % 3
            rs = u % 2

            @pl.when(i >= 1)
            def _():
                _drain_scatters((u - 1) % 3, (u - 1) % 2)

            @pl.when(i + 2 <= NWIN - 1)
            def _():
                pltpu.async_copy(cwin.at[win0 + i + 2],
                                 cb.at[(u + 2) % 3], semc)

            @pl.when(i + 1 <= NWIN - 1)
            def _():
                pltpu.make_async_copy(cwin.at[win0 + i + 1],
                                      cb.at[(u + 1) % 3], semc).wait()
                _fire_gathers((u + 1) % 3, (u + 1) % 2)

            _drain_gathers(cs, rs)
            for k in range(WIN_R):
                for g in range(8):
                    v = cb[cs, WIN_R + k, g * 16:(g + 1) * 16]
                    t = v - base_node
                    inr = (t >= 0) & (t < HALF)
                    trash = HALF + (v & TRASH_MASK)
                    cb[cs, WIN_R + k, g * 16:(g + 1) * 16] = jnp.where(
                        inr, t, trash)
            for k in range(WIN_R):
                rbase = rs * WIN_E + k * 128

                def _scale16(g, carry2, _rbase=rbase, _k=k):
                    wvec = plsc.bitcast(
                        cb[cs, 2 * WIN_R + _k, pl.ds(g * 16, 16)],
                        jnp.float32)
                    for j2 in range(16):
                        e = _rbase + g * 16 + j2
                        wgt = wvec[j2]
                        rows[e, 0:16] = rows[e, 0:16] * wgt
                        rows[e, 16:32] = rows[e, 16:32] * wgt
                    return carry2

                lax.fori_loop(0, 8, _scale16, 0)
            _fire_scatters(cs, rs)
        return carry

    lax.fori_loop(0, NWIN // 6, _super, 0)
    _drain_scatters((NWIN - 1) % 3, (NWIN - 1) % 2)
    plsc.subcore_barrier()

    wb_src = pl.multiple_of(s * SLAB, 8)
    wb_dst = pl.multiple_of(base_node + s * SLAB, 8)

    @pl.when(s < 15)
    def _wb_main():
        pltpu.sync_copy(acc.at[pl.ds(wb_src, SLAB)],
                        out.at[pl.ds(wb_dst, SLAB)])

    @pl.when(s == 15)
    def _wb_last():
        pltpu.sync_copy(acc.at[pl.ds(wb_src, SLAB_LAST)],
                        out.at[pl.ds(wb_dst, SLAB_LAST)])


@functools.lru_cache(maxsize=1)
def _make_step():
  return pl.kernel(
    _step_body,
    out_type=jax.ShapeDtypeStruct((N, D), jnp.float32),
    mesh=plsc.VectorSubcoreMesh(core_axis_name="c", subcore_axis_name="s",
                                num_cores=2, num_subcores=NS),
    scratch_types=[
        pltpu.VMEM((3, 3 * WIN_R, 128), jnp.int32),
        pltpu.VMEM((RBUF, D), jnp.float32),
        pltpu.VMEM_SHARED((ACC_ROWS, D), jnp.float32),
        pltpu.SemaphoreType.DMA,
        pltpu.SemaphoreType.DMA,
        pltpu.SemaphoreType.DMA,
    ],
    compiler_params=pltpu.CompilerParams(use_tc_tiling_on_sc=False,
                                         needs_layout_passes=False),
  )


def _acc_body(a, b, o):
    o[...] = a[...] + b[...]


_acc_add = pl.pallas_call(
    _acc_body,
    grid=(50,),
    in_specs=[pl.BlockSpec((2000, D), lambda i: (i, 0))] * 2,
    out_specs=pl.BlockSpec((2000, D), lambda i: (i, 0)),
    out_shape=jax.ShapeDtypeStruct((N, D), jnp.float32),
)


def _scale_body(a, o):
    o[...] = a[...] * 0.25


_scale_q = pl.pallas_call(
    _scale_body,
    grid=(50,),
    in_specs=[pl.BlockSpec((2000, D), lambda i: (i, 0))],
    out_specs=pl.BlockSpec((2000, D), lambda i: (i, 0)),
    out_shape=jax.ShapeDtypeStruct((N, D), jnp.float32),
)


def kernel(playlist_w, track_w, edge_weight, edge_index):
    emb0 = jnp.concatenate([playlist_w, track_w], axis=0)
    src = edge_index[0]
    dst = edge_index[1]
    pad = E_PAD - E
    padidx = (jnp.arange(pad, dtype=jnp.int32) * 61) % N
    src3 = jnp.concatenate([src, padidx]).reshape(NWTOT, WIN_R, 128)
    dst3 = jnp.concatenate([dst, padidx]).reshape(NWTOT, WIN_R, 128)
    w3 = lax.bitcast_convert_type(
        jnp.concatenate([edge_weight, jnp.zeros((pad,), jnp.float32)]),
        jnp.int32).reshape(NWTOT, WIN_R, 128)
    cwin = jnp.concatenate([src3, dst3, w3], axis=1)

    step = _make_step()

    def _layer(i, carry):
        emb, ssum = carry
        e = step(emb, cwin)
        return (e, _acc_add(ssum, e))

    _, ssum = lax.fori_loop(0, 3, _layer, (emb0, emb0))
    final = _scale_q(ssum)
    return final[:NP_], final[NP_:]
